# baseline (speedup 1.0000x reference)
import functools

import jax
import jax.numpy as jnp
from jax import lax
from jax.experimental import pallas as pl
from jax.experimental.pallas import tpu as pltpu
from jax.experimental.pallas import tpu_sc as plsc

F = 26
V = 100000
E = 32
EPS = 1e-5
_GROUPS = (26,)

try:
    _info = plsc.get_sparse_core_info()
    _NC, _NS = _info.num_cores, _info.num_subcores
except Exception:
    _NC, _NS = 2, 16
_NW = _NC * _NS
_CHUNK = 128


def _emb_gather(idxe_hbm, emb_hbm, out_emb, idxe_v, buf_v, sem_e, wid,
                nechunk, nhalf):
    half = nechunk // nhalf
    for h in range(nhalf):
        pltpu.sync_copy(idxe_hbm.at[wid].at[pl.ds(h * half, half)], idxe_v)

        def _fire(j, _):
            pltpu.async_copy(emb_hbm.at[idxe_v.at[j]], buf_v.at[j], sem_e)
            return 0

        lax.fori_loop(0, half, _fire, 0)

        def _drain(j, _):
            pltpu.make_async_copy(emb_hbm.at[pl.ds(0, _CHUNK)],
                                  buf_v.at[0], sem_e).wait()
            return 0

        lax.fori_loop(0, half, _drain, 0)
        pltpu.sync_copy(buf_v, out_emb.at[wid].at[pl.ds(h * half, half)])


def _sc_gather_fm_body(idxr_hbm, idxe_hbm, emb_hbm, fm_hbm, out_emb, out_fm,
                       idxr_v, idxe_v, buf_v, fm_v, sem_e, sem_f,
                       *, nrchunk, nechunk, nhalf):
    wid = lax.axis_index("s") * _NC + lax.axis_index("c")
    pltpu.sync_copy(idxr_hbm.at[wid], idxr_v)
    fm_copies = []
    for c in range(nrchunk):
        fm_copies.append(pltpu.async_copy(
            fm_hbm.at[idxr_v.at[c]], fm_v.at[c], sem_f))
    _emb_gather(idxe_hbm, emb_hbm, out_emb, idxe_v, buf_v, sem_e, wid,
                nechunk, nhalf)
    for cp in fm_copies:
        cp.wait()
    pltpu.sync_copy(fm_v, out_fm.at[wid])


def _sc_gather_body(idxe_hbm, emb_hbm, out_emb, idxe_v, buf_v, sem_e,
                    *, nechunk, nhalf):
    wid = lax.axis_index("s") * _NC + lax.axis_index("c")
    _emb_gather(idxe_hbm, emb_hbm, out_emb, idxe_v, buf_v, sem_e, wid,
                nechunk, nhalf)


def _sc_gather(idxe, flat_e, idxr=None, flat_fm=None):
    nechunk = idxe.shape[1]
    nhalf = 2
    mesh = plsc.VectorSubcoreMesh(core_axis_name="c", subcore_axis_name="s")
    emb_scratch = [
        pltpu.VMEM((nechunk // nhalf, _CHUNK), jnp.int32),
        pltpu.VMEM((nechunk // nhalf, _CHUNK), jnp.float32),
        pltpu.SemaphoreType.DMA,
    ]
    if idxr is None:
        kern = pl.kernel(
            functools.partial(_sc_gather_body, nechunk=nechunk, nhalf=nhalf),
            mesh=mesh,
            out_type=jax.ShapeDtypeStruct((_NW, nechunk, _CHUNK),
                                          jnp.float32),
            scratch_types=emb_scratch,
            compiler_params=pltpu.CompilerParams(use_tc_tiling_on_sc=False),
        )
        return kern(idxe, flat_e)
    nrchunk = idxr.shape[1]
    kern = pl.kernel(
        functools.partial(_sc_gather_fm_body, nrchunk=nrchunk,
                          nechunk=nechunk, nhalf=nhalf),
        mesh=mesh,
        out_type=[
            jax.ShapeDtypeStruct((_NW, nechunk, _CHUNK), jnp.float32),
            jax.ShapeDtypeStruct((_NW, nrchunk, _CHUNK), jnp.float32),
        ],
        scratch_types=[
            pltpu.VMEM((nrchunk, _CHUNK), jnp.int32),
            emb_scratch[0],
            emb_scratch[1],
            pltpu.VMEM((nrchunk, _CHUNK), jnp.float32),
            pltpu.SemaphoreType.DMA,
            pltpu.SemaphoreType.DMA,
        ],
        compiler_params=pltpu.CompilerParams(use_tc_tiling_on_sc=False),
    )
    return kern(idxr, idxe, flat_e, flat_fm)


def _mlp_body(*refs):
    ng = len(_GROUPS)
    xs = refs[:ng]
    pars = refs[ng:2 * ng]
    (fmv_ref, w2_ref, wf_ref, b1_ref, g1_ref, be1_ref, rm1_ref, rv1_ref,
     b2_ref, g2_ref, be2_ref, rm2_ref, rv2_ref, bf_ref) = refs[2 * ng:
                                                              2 * ng + 14]
    w1s = refs[2 * ng + 14:3 * ng + 14]
    o_ref = refs[3 * ng + 14]

    z1 = b1_ref[...]
    s = jnp.zeros((xs[0].shape[0], E), jnp.float32)
    ss = jnp.zeros((xs[0].shape[0], E), jnp.float32)
    for x_ref, p_ref, w1_ref, fg in zip(xs, pars, w1s, _GROUPS):
        u = lax.bitcast_convert_type(x_ref[...], jnp.uint32)
        hi = lax.bitcast_convert_type(u & jnp.uint32(0xFFFF0000), jnp.float32)
        lo = lax.bitcast_convert_type(u << 16, jnp.float32)
        x = jnp.where(p_ref[...] > 0.5, hi, lo)
        z1 = z1 + jnp.dot(x, w1_ref[...],
                          preferred_element_type=jnp.float32)
        r = lax.broadcasted_iota(jnp.int32, (fg * E, E), 0) % E
        c = lax.broadcasted_iota(jnp.int32, (fg * E, E), 1)
        a_mat = (r == c).astype(jnp.float32)
        s = s + jnp.dot(x, a_mat, preferred_element_type=jnp.float32)
        ss = ss + jnp.dot(x * x, a_mat, preferred_element_type=jnp.float32)
    a1 = g1_ref[...] * lax.rsqrt(rv1_ref[...] + EPS)
    h1 = jnp.maximum(z1, 0.0) * a1 + (be1_ref[...] - rm1_ref[...] * a1)
    z2 = jnp.dot(h1, w2_ref[...], preferred_element_type=jnp.float32) + b2_ref[...]
    a2 = g2_ref[...] * lax.rsqrt(rv2_ref[...] + EPS)
    h2 = jnp.maximum(z2, 0.0) * a2 + (be2_ref[...] - rm2_ref[...] * a2)
    o = jnp.sum(h2 * wf_ref[...], axis=1, keepdims=True)
    fm2 = 0.5 * jnp.sum(s * s - ss, axis=1, keepdims=True)
    fm1 = jnp.sum(fmv_ref[...], axis=1, keepdims=True)
    logit = o + bf_ref[...] + fm1 + fm2
    o_ref[...] = 1.0 / (1.0 + jnp.exp(-logit))


def _tc_mlp(dnns, pars, fmv, w1s, w2t, wf, b1, g1, be1, rm1, rv1, b2, g2,
            be2, rm2, rv2, bf):
    b = dnns[0].shape[0]
    h1 = w2t.shape[0]
    h2 = w2t.shape[1]
    bt = 512
    grid = (b // bt,)
    full = lambda shape: pl.BlockSpec(shape, lambda i: (0, 0))
    in_specs = (
        [pl.BlockSpec((bt, fg * E), lambda i: (i, 0)) for fg in _GROUPS]
        + [pl.BlockSpec((bt, fg * E), lambda i: (i, 0)) for fg in _GROUPS]
        + [pl.BlockSpec((bt, F), lambda i: (i, 0)),
           full((h1, h2)), full((1, h2)),
           full((1, h1)), full((1, h1)), full((1, h1)), full((1, h1)),
           full((1, h1)),
           full((1, h2)), full((1, h2)), full((1, h2)), full((1, h2)),
           full((1, h2)),
           full((1, 1))]
        + [full((fg * E, h1)) for fg in _GROUPS]
    )
    return pl.pallas_call(
        _mlp_body,
        grid=grid,
        in_specs=in_specs,
        out_specs=pl.BlockSpec((bt, 1), lambda i: (i, 0)),
        out_shape=jax.ShapeDtypeStruct((b, 1), jnp.float32),
    )(*dnns, *pars, fmv, w2t, wf, b1, g1, be1, rm1, rv1, b2, g2, be2, rm2,
      rv2, bf, *w1s)


def kernel(x, emb_tables, fm_tables, W1, b1, g1, be1, rm1, rv1,
           W2, b2, g2, be2, rm2, rv2, Wf, bf):
    b = x.shape[0]
    n_rows = b * F
    xi = x.astype(jnp.int32)
    idx = xi + (jnp.arange(F, dtype=jnp.int32) * V)[None, :]
    idxr = idx.reshape(_NW, n_rows // (_NW * _CHUNK), _CHUNK)
    flat_fm = fm_tables.reshape(F * V)

    h1 = W1.shape[0]
    w1p = jnp.transpose(W1.reshape(h1, E, F), (2, 1, 0)).reshape(F * E, h1)

    dnns, w1s, pars = [], [], []
    fm_rows = None
    f0 = 0
    for g, fg in enumerate(_GROUPS):
        f1 = f0 + fg
        idxe = (xi[:, f0:f1, None]
                + ((jnp.arange(fg, dtype=jnp.int32) * E)[:, None]
                   + jnp.arange(E, dtype=jnp.int32)[None, :])[None] * V)
        idxe = (idxe >> 1).reshape(_NW, b * fg * E // (_NW * _CHUNK), _CHUNK)
        flat_bf = jnp.transpose(emb_tables[f0:f1], (0, 2, 1)).astype(
            jnp.bfloat16).reshape(fg * E * V // 2, 2)
        flat_e = lax.bitcast_convert_type(flat_bf, jnp.float32)
        parg = jnp.broadcast_to(
            (xi[:, f0:f1] & 1).astype(jnp.float32)[:, :, None], (b, fg, E))
        pars.append(parg.reshape(b, fg * E))
        if g == 0:
            emb_rows, fm_rows = _sc_gather(idxe, flat_e, idxr, flat_fm)
        else:
            emb_rows = _sc_gather(idxe, flat_e)
        dnns.append(emb_rows.reshape(b, fg * E))
        w1s.append(w1p[f0 * E:f1 * E])
        f0 = f1

    fmv = fm_rows.reshape(b, F)
    w2t = W2.T
    r2 = lambda v: v.reshape(1, -1)
    return _tc_mlp(dnns, pars, fmv, w1s, w2t, r2(Wf), r2(b1), r2(g1),
                   r2(be1), r2(rm1), r2(rv1), r2(b2), r2(g2), r2(be2),
                   r2(rm2), r2(rv2), bf.reshape(1, 1))


# speedup vs baseline: 2.4388x; 2.4388x over previous
import functools

import jax
import jax.numpy as jnp
from jax import lax
from jax.experimental import pallas as pl
from jax.experimental.pallas import tpu as pltpu
from jax.experimental.pallas import tpu_sc as plsc

F = 26
V = 100000
E = 32
EPS = 1e-5
_GROUPS = (26,)

try:
    _info = plsc.get_sparse_core_info()
    _NC, _NS = _info.num_cores, _info.num_subcores
except Exception:
    _NC, _NS = 2, 16
_NW = _NC * _NS
_CHUNK = 128


def _emb_gather(idxe_hbm, emb_hbm, out_emb, idxe_v, buf_v, sem_e, wid,
                nechunk, nhalf):
    half = nechunk // nhalf
    for h in range(nhalf):
        pltpu.sync_copy(idxe_hbm.at[wid].at[pl.ds(h * half, half)], idxe_v)

        def _fire(j, _):
            pltpu.async_copy(emb_hbm.at[idxe_v.at[j]], buf_v.at[j], sem_e)
            return 0

        lax.fori_loop(0, half, _fire, 0)

        def _drain(j, _):
            pltpu.make_async_copy(emb_hbm.at[pl.ds(0, _CHUNK)],
                                  buf_v.at[0], sem_e).wait()
            return 0

        lax.fori_loop(0, half, _drain, 0)
        pltpu.sync_copy(buf_v, out_emb.at[wid].at[pl.ds(h * half, half)])


def _sc_gather_fm_body(idxr_hbm, idxe_hbm, emb_hbm, fm_hbm, out_emb, out_fm,
                       idxr_v, idxe_v, buf_v, fm_v, sem_e, sem_f,
                       *, nrchunk, nechunk, nhalf):
    wid = lax.axis_index("s") * _NC + lax.axis_index("c")
    pltpu.sync_copy(idxr_hbm.at[wid], idxr_v)
    fm_copies = []
    for c in range(nrchunk):
        fm_copies.append(pltpu.async_copy(
            fm_hbm.at[idxr_v.at[c]], fm_v.at[c], sem_f))
    _emb_gather(idxe_hbm, emb_hbm, out_emb, idxe_v, buf_v, sem_e, wid,
                nechunk, nhalf)
    for cp in fm_copies:
        cp.wait()
    pltpu.sync_copy(fm_v, out_fm.at[wid])


def _sc_gather_body(idxe_hbm, emb_hbm, out_emb, idxe_v, buf_v, sem_e,
                    *, nechunk, nhalf):
    wid = lax.axis_index("s") * _NC + lax.axis_index("c")
    _emb_gather(idxe_hbm, emb_hbm, out_emb, idxe_v, buf_v, sem_e, wid,
                nechunk, nhalf)


def _sc_gather(idxe, flat_e, idxr=None, flat_fm=None):
    nechunk = idxe.shape[1]
    nhalf = 2
    mesh = plsc.VectorSubcoreMesh(core_axis_name="c", subcore_axis_name="s")
    emb_scratch = [
        pltpu.VMEM((nechunk // nhalf, _CHUNK), jnp.int32),
        pltpu.VMEM((nechunk // nhalf, _CHUNK), jnp.float32),
        pltpu.SemaphoreType.DMA,
    ]
    if idxr is None:
        kern = pl.kernel(
            functools.partial(_sc_gather_body, nechunk=nechunk, nhalf=nhalf),
            mesh=mesh,
            out_type=jax.ShapeDtypeStruct((_NW, nechunk, _CHUNK),
                                          jnp.float32),
            scratch_types=emb_scratch,
            compiler_params=pltpu.CompilerParams(use_tc_tiling_on_sc=False),
        )
        return kern(idxe, flat_e)
    nrchunk = idxr.shape[1]
    kern = pl.kernel(
        functools.partial(_sc_gather_fm_body, nrchunk=nrchunk,
                          nechunk=nechunk, nhalf=nhalf),
        mesh=mesh,
        out_type=[
            jax.ShapeDtypeStruct((_NW, nechunk, _CHUNK), jnp.float32),
            jax.ShapeDtypeStruct((_NW, nrchunk, _CHUNK), jnp.float32),
        ],
        scratch_types=[
            pltpu.VMEM((nrchunk, _CHUNK), jnp.int32),
            emb_scratch[0],
            emb_scratch[1],
            pltpu.VMEM((nrchunk, _CHUNK), jnp.float32),
            pltpu.SemaphoreType.DMA,
            pltpu.SemaphoreType.DMA,
        ],
        compiler_params=pltpu.CompilerParams(use_tc_tiling_on_sc=False),
    )
    return kern(idxr, idxe, flat_e, flat_fm)


def _mlp_body(*refs):
    ng = len(_GROUPS)
    xs = refs[:ng]
    pars = refs[ng:2 * ng]
    (fmv_ref, w2_ref, wf_ref, b1_ref, g1_ref, be1_ref, rm1_ref, rv1_ref,
     b2_ref, g2_ref, be2_ref, rm2_ref, rv2_ref, bf_ref) = refs[2 * ng:
                                                              2 * ng + 14]
    w1s = refs[2 * ng + 14:3 * ng + 14]
    o_ref = refs[3 * ng + 14]

    z1 = b1_ref[...]
    s = jnp.zeros((xs[0].shape[0], E), jnp.float32)
    ss = jnp.zeros((xs[0].shape[0], E), jnp.float32)
    for x_ref, p_ref, w1_ref, fg in zip(xs, pars, w1s, _GROUPS):
        u = lax.bitcast_convert_type(x_ref[...], jnp.uint32)
        hi = lax.bitcast_convert_type(u & jnp.uint32(0xFFFF0000), jnp.float32)
        lo = lax.bitcast_convert_type(u << 16, jnp.float32)
        x = jnp.where(p_ref[...] > 0.5, hi, lo)
        z1 = z1 + jnp.dot(x, w1_ref[...],
                          preferred_element_type=jnp.float32)
        r = lax.broadcasted_iota(jnp.int32, (fg * E, E), 0) % E
        c = lax.broadcasted_iota(jnp.int32, (fg * E, E), 1)
        a_mat = (r == c).astype(jnp.float32)
        s = s + jnp.dot(x, a_mat, preferred_element_type=jnp.float32)
        ss = ss + jnp.dot(x * x, a_mat, preferred_element_type=jnp.float32)
    a1 = g1_ref[...] * lax.rsqrt(rv1_ref[...] + EPS)
    h1 = jnp.maximum(z1, 0.0) * a1 + (be1_ref[...] - rm1_ref[...] * a1)
    z2 = jnp.dot(h1, w2_ref[...], preferred_element_type=jnp.float32) + b2_ref[...]
    a2 = g2_ref[...] * lax.rsqrt(rv2_ref[...] + EPS)
    h2 = jnp.maximum(z2, 0.0) * a2 + (be2_ref[...] - rm2_ref[...] * a2)
    o = jnp.sum(h2 * wf_ref[...], axis=1, keepdims=True)
    fm2 = 0.5 * jnp.sum(s * s - ss, axis=1, keepdims=True)
    fm1 = jnp.sum(fmv_ref[...], axis=1, keepdims=True)
    logit = o + bf_ref[...] + fm1 + fm2
    o_ref[...] = 1.0 / (1.0 + jnp.exp(-logit))


def _tc_mlp(dnns, pars, fmv, w1s, w2t, wf, b1, g1, be1, rm1, rv1, b2, g2,
            be2, rm2, rv2, bf):
    b = dnns[0].shape[0]
    h1 = w2t.shape[0]
    h2 = w2t.shape[1]
    bt = 512
    grid = (b // bt,)
    full = lambda shape: pl.BlockSpec(shape, lambda i: (0, 0))
    in_specs = (
        [pl.BlockSpec((bt, fg * E), lambda i: (i, 0)) for fg in _GROUPS]
        + [pl.BlockSpec((bt, fg * E), lambda i: (i, 0)) for fg in _GROUPS]
        + [pl.BlockSpec((bt, F), lambda i: (i, 0)),
           full((h1, h2)), full((1, h2)),
           full((1, h1)), full((1, h1)), full((1, h1)), full((1, h1)),
           full((1, h1)),
           full((1, h2)), full((1, h2)), full((1, h2)), full((1, h2)),
           full((1, h2)),
           full((1, 1))]
        + [full((fg * E, h1)) for fg in _GROUPS]
    )
    return pl.pallas_call(
        _mlp_body,
        grid=grid,
        in_specs=in_specs,
        out_specs=pl.BlockSpec((bt, 1), lambda i: (i, 0)),
        out_shape=jax.ShapeDtypeStruct((b, 1), jnp.float32),
    )(*dnns, *pars, fmv, w2t, wf, b1, g1, be1, rm1, rv1, b2, g2, be2, rm2,
      rv2, bf, *w1s)


def kernel(x, emb_tables, fm_tables, W1, b1, g1, be1, rm1, rv1,
           W2, b2, g2, be2, rm2, rv2, Wf, bf):
    b = x.shape[0]
    n_rows = b * F
    xi = x.astype(jnp.int32)
    idx = xi + (jnp.arange(F, dtype=jnp.int32) * V)[None, :]
    idxr = idx.reshape(_NW, n_rows // (_NW * _CHUNK), _CHUNK)
    flat_fm = fm_tables.reshape(F * V)

    h1 = W1.shape[0]
    w1p = jnp.transpose(W1.reshape(h1, E, F), (2, 1, 0)).reshape(F * E, h1)

    dnns, w1s, pars = [], [], []
    fm_rows = None
    f0 = 0
    for g, fg in enumerate(_GROUPS):
        f1 = f0 + fg
        idxe = (xi[:, f0:f1, None]
                + ((jnp.arange(fg, dtype=jnp.int32) * E)[:, None]
                   + jnp.arange(E, dtype=jnp.int32)[None, :])[None] * V)
        idxe = (idxe >> 1).reshape(_NW, b * fg * E // (_NW * _CHUNK), _CHUNK)
        t = jnp.transpose(emb_tables[f0:f1], (0, 2, 1))
        tb = lax.bitcast_convert_type(t, jnp.uint32)
        tr = (tb + jnp.uint32(0x7FFF) + ((tb >> 16) & jnp.uint32(1))) >> 16
        packed = tr[:, :, 0::2] | (tr[:, :, 1::2] << 16)
        flat_e = lax.bitcast_convert_type(packed, jnp.float32).reshape(
            fg * E * V // 2)
        parg = jnp.broadcast_to(
            (xi[:, f0:f1] & 1).astype(jnp.float32)[:, :, None], (b, fg, E))
        pars.append(parg.reshape(b, fg * E))
        if g == 0:
            emb_rows, fm_rows = _sc_gather(idxe, flat_e, idxr, flat_fm)
        else:
            emb_rows = _sc_gather(idxe, flat_e)
        dnns.append(emb_rows.reshape(b, fg * E))
        w1s.append(w1p[f0 * E:f1 * E])
        f0 = f1

    fmv = fm_rows.reshape(b, F)
    w2t = W2.T
    r2 = lambda v: v.reshape(1, -1)
    return _tc_mlp(dnns, pars, fmv, w1s, w2t, r2(Wf), r2(b1), r2(g1),
                   r2(be1), r2(rm1), r2(rv1), r2(b2), r2(g2), r2(be2),
                   r2(rm2), r2(rv2), bf.reshape(1, 1))


# speedup vs baseline: 34.5608x; 14.1713x over previous
import functools

import jax
import jax.numpy as jnp
from jax import lax
from jax.experimental import pallas as pl
from jax.experimental.pallas import tpu as pltpu
from jax.experimental.pallas import tpu_sc as plsc

F = 26
V = 100000
E = 32
EPS = 1e-5
_GROUPS = (26,)

try:
    _info = plsc.get_sparse_core_info()
    _NC, _NS = _info.num_cores, _info.num_subcores
except Exception:
    _NC, _NS = 2, 16
_NW = _NC * _NS
_CHUNK = 128


def _emb_gather(idxe_hbm, emb_hbm, out_emb, idxe_v, buf_v, sem_e, wid,
                nechunk, nhalf):
    half = nechunk // nhalf
    for h in range(nhalf):
        pltpu.sync_copy(idxe_hbm.at[wid].at[pl.ds(h * half, half)], idxe_v)

        def _fire(j, _):
            pltpu.async_copy(emb_hbm.at[idxe_v.at[j]], buf_v.at[j], sem_e)
            return 0

        lax.fori_loop(0, half, _fire, 0)

        def _drain(j, _):
            pltpu.make_async_copy(emb_hbm.at[pl.ds(0, _CHUNK)],
                                  buf_v.at[0], sem_e).wait()
            return 0

        lax.fori_loop(0, half, _drain, 0)
        pltpu.sync_copy(buf_v, out_emb.at[wid].at[pl.ds(h * half, half)])


def _sc_gather_fm_body(idxr_hbm, idxe_hbm, emb_hbm, fm_hbm, out_emb, out_fm,
                       idxr_v, idxe_v, buf_v, fm_v, sem_e, sem_f,
                       *, nrchunk, nechunk, nhalf):
    wid = lax.axis_index("s") * _NC + lax.axis_index("c")
    pltpu.sync_copy(idxr_hbm.at[wid], idxr_v)
    fm_copies = []
    for c in range(nrchunk):
        fm_copies.append(pltpu.async_copy(
            fm_hbm.at[idxr_v.at[c]], fm_v.at[c], sem_f))
    _emb_gather(idxe_hbm, emb_hbm, out_emb, idxe_v, buf_v, sem_e, wid,
                nechunk, nhalf)
    for cp in fm_copies:
        cp.wait()
    pltpu.sync_copy(fm_v, out_fm.at[wid])


def _sc_gather_body(idxe_hbm, emb_hbm, out_emb, idxe_v, buf_v, sem_e,
                    *, nechunk, nhalf):
    wid = lax.axis_index("s") * _NC + lax.axis_index("c")
    _emb_gather(idxe_hbm, emb_hbm, out_emb, idxe_v, buf_v, sem_e, wid,
                nechunk, nhalf)


def _sc_gather(idxe, flat_e, idxr=None, flat_fm=None):
    nechunk = idxe.shape[1]
    nhalf = 2
    mesh = plsc.VectorSubcoreMesh(core_axis_name="c", subcore_axis_name="s")
    emb_scratch = [
        pltpu.VMEM((nechunk // nhalf, _CHUNK), jnp.int32),
        pltpu.VMEM((nechunk // nhalf, _CHUNK), jnp.float32),
        pltpu.SemaphoreType.DMA,
    ]
    if idxr is None:
        kern = pl.kernel(
            functools.partial(_sc_gather_body, nechunk=nechunk, nhalf=nhalf),
            mesh=mesh,
            out_type=jax.ShapeDtypeStruct((_NW, nechunk, _CHUNK),
                                          jnp.float32),
            scratch_types=emb_scratch,
            compiler_params=pltpu.CompilerParams(use_tc_tiling_on_sc=False),
        )
        return kern(idxe, flat_e)
    nrchunk = idxr.shape[1]
    kern = pl.kernel(
        functools.partial(_sc_gather_fm_body, nrchunk=nrchunk,
                          nechunk=nechunk, nhalf=nhalf),
        mesh=mesh,
        out_type=[
            jax.ShapeDtypeStruct((_NW, nechunk, _CHUNK), jnp.float32),
            jax.ShapeDtypeStruct((_NW, nrchunk, _CHUNK), jnp.float32),
        ],
        scratch_types=[
            pltpu.VMEM((nrchunk, _CHUNK), jnp.int32),
            emb_scratch[0],
            emb_scratch[1],
            pltpu.VMEM((nrchunk, _CHUNK), jnp.float32),
            pltpu.SemaphoreType.DMA,
            pltpu.SemaphoreType.DMA,
        ],
        compiler_params=pltpu.CompilerParams(use_tc_tiling_on_sc=False),
    )
    return kern(idxr, idxe, flat_e, flat_fm)


def _mlp_body(*refs):
    ng = len(_GROUPS)
    xs = refs[:ng]
    (fmv_ref, w2_ref, wf_ref, b1_ref, g1_ref, be1_ref, rm1_ref, rv1_ref,
     b2_ref, g2_ref, be2_ref, rm2_ref, rv2_ref, bf_ref) = refs[ng:ng + 14]
    w1s = refs[ng + 14:2 * ng + 14]
    o_ref = refs[2 * ng + 14]

    z1 = b1_ref[...]
    s = jnp.zeros((xs[0].shape[0], E), jnp.float32)
    ss = jnp.zeros((xs[0].shape[0], E), jnp.float32)
    for x_ref, w1_ref, fg in zip(xs, w1s, _GROUPS):
        x = x_ref[...]
        z1 = z1 + jnp.dot(x, w1_ref[...],
                          preferred_element_type=jnp.float32)
        r = lax.broadcasted_iota(jnp.int32, (fg * E, E), 0) % E
        c = lax.broadcasted_iota(jnp.int32, (fg * E, E), 1)
        a_mat = (r == c).astype(jnp.float32)
        s = s + jnp.dot(x, a_mat, preferred_element_type=jnp.float32)
        ss = ss + jnp.dot(x * x, a_mat, preferred_element_type=jnp.float32)
    a1 = g1_ref[...] * lax.rsqrt(rv1_ref[...] + EPS)
    h1 = jnp.maximum(z1, 0.0) * a1 + (be1_ref[...] - rm1_ref[...] * a1)
    z2 = jnp.dot(h1, w2_ref[...], preferred_element_type=jnp.float32) + b2_ref[...]
    a2 = g2_ref[...] * lax.rsqrt(rv2_ref[...] + EPS)
    h2 = jnp.maximum(z2, 0.0) * a2 + (be2_ref[...] - rm2_ref[...] * a2)
    o = jnp.sum(h2 * wf_ref[...], axis=1, keepdims=True)
    fm2 = 0.5 * jnp.sum(s * s - ss, axis=1, keepdims=True)
    fm1 = jnp.sum(fmv_ref[...], axis=1, keepdims=True)
    logit = o + bf_ref[...] + fm1 + fm2
    o_ref[...] = 1.0 / (1.0 + jnp.exp(-logit))


def _tc_mlp(dnns, fmv, w1s, w2t, wf, b1, g1, be1, rm1, rv1, b2, g2, be2,
            rm2, rv2, bf):
    b = dnns[0].shape[0]
    h1 = w2t.shape[0]
    h2 = w2t.shape[1]
    bt = 512
    grid = (b // bt,)
    full = lambda shape: pl.BlockSpec(shape, lambda i: (0, 0))
    in_specs = (
        [pl.BlockSpec((bt, fg * E), lambda i: (i, 0)) for fg in _GROUPS]
        + [pl.BlockSpec((bt, F), lambda i: (i, 0)),
           full((h1, h2)), full((1, h2)),
           full((1, h1)), full((1, h1)), full((1, h1)), full((1, h1)),
           full((1, h1)),
           full((1, h2)), full((1, h2)), full((1, h2)), full((1, h2)),
           full((1, h2)),
           full((1, 1))]
        + [full((fg * E, h1)) for fg in _GROUPS]
    )
    return pl.pallas_call(
        _mlp_body,
        grid=grid,
        in_specs=in_specs,
        out_specs=pl.BlockSpec((bt, 1), lambda i: (i, 0)),
        out_shape=jax.ShapeDtypeStruct((b, 1), jnp.float32),
    )(*dnns, fmv, w2t, wf, b1, g1, be1, rm1, rv1, b2, g2, be2, rm2, rv2,
      bf, *w1s)


def kernel(x, emb_tables, fm_tables, W1, b1, g1, be1, rm1, rv1,
           W2, b2, g2, be2, rm2, rv2, Wf, bf):
    b = x.shape[0]
    n_rows = b * F
    xi = x.astype(jnp.int32)
    idx = xi + (jnp.arange(F, dtype=jnp.int32) * V)[None, :]
    idxr = idx.reshape(_NW, n_rows // (_NW * _CHUNK), _CHUNK)
    flat_fm = fm_tables.reshape(F * V)

    h1 = W1.shape[0]
    w1p = jnp.transpose(W1.reshape(h1, E, F), (2, 1, 0)).reshape(F * E, h1)

    dnns, w1s = [], []
    fm_rows = None
    f0 = 0
    for g, fg in enumerate(_GROUPS):
        f1 = f0 + fg
        idxe = (xi[:, f0:f1, None]
                + ((jnp.arange(fg, dtype=jnp.int32) * E)[:, None]
                   + jnp.arange(E, dtype=jnp.int32)[None, :])[None] * V)
        idxe = idxe.reshape(_NW, b * fg * E // (_NW * _CHUNK), _CHUNK)
        flat_e = jnp.transpose(emb_tables[f0:f1], (0, 2, 1)).reshape(
            fg * E * V)
        if g == 0:
            emb_rows, fm_rows = _sc_gather(idxe, flat_e, idxr, flat_fm)
        else:
            emb_rows = _sc_gather(idxe, flat_e)
        dnns.append(emb_rows.reshape(b, fg * E))
        w1s.append(w1p[f0 * E:f1 * E])
        f0 = f1

    fmv = fm_rows.reshape(b, F)
    w2t = W2.T
    r2 = lambda v: v.reshape(1, -1)
    return _tc_mlp(dnns, fmv, w1s, w2t, r2(Wf), r2(b1), r2(g1), r2(be1),
                   r2(rm1), r2(rv1), r2(b2), r2(g2), r2(be2), r2(rm2),
                   r2(rv2), bf.reshape(1, 1))
